# baseline (speedup 1.0000x reference)
import jax
import jax.numpy as jnp
from jax.experimental import pallas as pl
from jax.experimental.pallas import tpu as pltpu

N = 100000
D = 128
BLK = 4000
NB = N // BLK
G = BLK // 32


def _sigmoid(x):
    return 1.0 / (1.0 + jnp.exp(-x))


def _row_to_col(x_row, eye):
    return jnp.sum(jnp.where(eye, x_row, 0.0), axis=1, keepdims=True)


def _col_to_row(x_col, eye):
    return jnp.sum(jnp.where(eye, x_col, 0.0), axis=0, keepdims=True)


def _body(mem_ref, wa_ref, ba_ref, w1_ref, b1c_ref, w2c_ref, b2_ref,
          wih_ref, bihc_ref, whh_ref, bhhc_ref,
          nll_ref, val_ref, state_ref,
          mem_sc, acc_sc, stat_sc):
    i = pl.program_id(0)

    @pl.when(i == 0)
    def _init():
        stat_sc[0] = -jnp.inf
        stat_sc[1] = 0.0
        acc_sc[...] = jnp.zeros_like(acc_sc)

    blk = mem_ref[...]
    mem_sc[pl.ds(i * BLK, BLK), :] = blk
    blk3 = blk.reshape(G, 32, D)

    wa3 = wa_ref[...].reshape(1, 1, D)
    l3 = jnp.sum(blk3 * wa3, axis=2) + ba_ref[0, 0]
    bm = jnp.max(l3)
    m_old = stat_sc[0]
    m_new = jnp.maximum(m_old, bm)
    scale = jnp.exp(m_old - m_new)
    w3 = jnp.exp(l3 - m_new)
    stat_sc[0] = m_new
    stat_sc[1] = stat_sc[1] * scale + jnp.sum(w3)
    wsum = jnp.sum(w3[:, :, None] * blk3, axis=0)
    acc_sc[...] = acc_sc[...] * scale + jnp.sum(wsum, axis=0, keepdims=True)

    @pl.when(i == NB - 1)
    def _epilogue():
        s1 = stat_sc[1]
        state_row = acc_sc[...] / s1

        eye = (jax.lax.broadcasted_iota(jnp.int32, (D, D), 0)
               == jax.lax.broadcasted_iota(jnp.int32, (D, D), 1))

        t = jnp.sum(w1_ref[...] * state_row, axis=1, keepdims=True)
        h = jnp.maximum(t + b1c_ref[...], 0.0)
        value = jnp.sum(w2c_ref[...] * h) + b2_ref[0, 0]

        state3 = state_row.reshape(1, 1, D)
        idx3 = (jax.lax.broadcasted_iota(jnp.int32, (G, 32), 0) * 32
                + jax.lax.broadcasted_iota(jnp.int32, (G, 32), 1))

        def body(j, carry):
            m2, s2, gmax, gidx = carry
            blk2 = mem_sc[pl.ds(j * BLK, BLK), :].reshape(G, 32, D)
            lg = jnp.sum(blk2 * state3, axis=2)
            bmax = jnp.max(lg)
            new_m = jnp.maximum(m2, bmax)
            s2 = s2 * jnp.exp(m2 - new_m) + jnp.sum(jnp.exp(lg - new_m))
            barg = jnp.min(jnp.where(lg == bmax, idx3, N))
            gidx = jnp.where(bmax > gmax, j * BLK + barg, gidx)
            gmax = jnp.maximum(gmax, bmax)
            return new_m, s2, gmax, gidx

        m2, s2, gmax, gidx = jax.lax.fori_loop(
            0, NB, body,
            (-jnp.inf, jnp.float32(0.0), -jnp.inf, jnp.int32(0)))
        lse = m2 + jnp.log(s2)
        nll_ref[...] = jnp.full((1, 1), lse - gmax, dtype=jnp.float32)
        val_ref[...] = jnp.full((1, 1), value, dtype=jnp.float32)

        g = (gidx // 8) * 8
        tile = mem_sc[pl.ds(g, 8), :]
        rows8 = jax.lax.broadcasted_iota(jnp.int32, (8, 1), 0)
        act_row = jnp.sum(jnp.where(rows8 == (gidx - g), tile, 0.0),
                          axis=0, keepdims=True)

        gi = jnp.sum(wih_ref[...] * act_row, axis=1, keepdims=True) + bihc_ref[...]
        gh = jnp.sum(whh_ref[...] * state_row, axis=1, keepdims=True) + bhhc_ref[...]
        i_r, i_z, i_n = gi[0:D], gi[D:2 * D], gi[2 * D:3 * D]
        h_r, h_z, h_n = gh[0:D], gh[D:2 * D], gh[2 * D:3 * D]
        r = _sigmoid(i_r + h_r)
        z = _sigmoid(i_z + h_z)
        n = jnp.tanh(i_n + r * h_n)
        state_col = _row_to_col(state_row, eye)
        new_col = (1.0 - z) * n + z * state_col
        state_ref[...] = _col_to_row(new_col, eye)


def kernel(mem, Wa, ba, W1, b1, W2, b2, W_ih, b_ih, W_hh, b_hh):
    ba2 = ba.reshape(1, 1)
    b1c = b1.reshape(D, 1)
    w2c = W2.reshape(D, 1)
    b2_2 = b2.reshape(1, 1)
    bihc = b_ih.reshape(3 * D, 1)
    bhhc = b_hh.reshape(3 * D, 1)

    const = lambda i: (0, 0)
    nll, val, st = pl.pallas_call(
        _body,
        grid=(NB,),
        in_specs=[
            pl.BlockSpec((BLK, D), lambda i: (i, 0)),
            pl.BlockSpec((1, D), const),
            pl.BlockSpec((1, 1), const),
            pl.BlockSpec((D, D), const),
            pl.BlockSpec((D, 1), const),
            pl.BlockSpec((D, 1), const),
            pl.BlockSpec((1, 1), const),
            pl.BlockSpec((3 * D, D), const),
            pl.BlockSpec((3 * D, 1), const),
            pl.BlockSpec((3 * D, D), const),
            pl.BlockSpec((3 * D, 1), const),
        ],
        out_specs=[
            pl.BlockSpec((1, 1), const),
            pl.BlockSpec((1, 1), const),
            pl.BlockSpec((1, D), const),
        ],
        out_shape=[
            jax.ShapeDtypeStruct((1, 1), jnp.float32),
            jax.ShapeDtypeStruct((1, 1), jnp.float32),
            jax.ShapeDtypeStruct((1, D), jnp.float32),
        ],
        scratch_shapes=[
            pltpu.VMEM((N, D), jnp.float32),
            pltpu.VMEM((1, D), jnp.float32),
            pltpu.SMEM((2,), jnp.float32),
        ],
        compiler_params=pltpu.CompilerParams(
            dimension_semantics=("arbitrary",),
            vmem_limit_bytes=64 * 1024 * 1024,
        ),
    )(mem, Wa, ba2, W1, b1c, w2c, b2_2, W_ih, bihc, W_hh, bhhc)
    return nll.reshape(()), val, st


# speedup vs baseline: 2.6004x; 2.6004x over previous
import jax
import jax.numpy as jnp
from jax import lax
from jax.experimental import pallas as pl
from jax.experimental.pallas import tpu as pltpu

N = 100000
D = 128
BLK = 10000
NB = N // BLK

_DOT_T = (((1,), (1,)), ((), ()))


def _sigmoid(x):
    return 1.0 / (1.0 + jnp.exp(-x))


def _bdot(a, b):
    return lax.dot_general(a.astype(jnp.bfloat16), b.astype(jnp.bfloat16),
                           _DOT_T, preferred_element_type=jnp.float32)


def _body(mem_ref, wa_ref, ba_ref, w1_ref, b1_ref, w2_ref, b2_ref,
          wih_ref, bih_ref, whh_ref, bhh_ref,
          nll_ref, val_ref, state_ref,
          mem_sc, acc_sc, stat_sc):
    i = pl.program_id(0)

    @pl.when(i == 0)
    def _init():
        stat_sc[0] = -jnp.inf
        stat_sc[1] = 0.0
        acc_sc[...] = jnp.zeros_like(acc_sc)

    blk = mem_ref[...]
    blk16 = blk.astype(jnp.bfloat16)
    mem_sc[pl.ds(i * BLK, BLK), :] = blk16

    l_row = _bdot(wa_ref[...], blk16) + ba_ref[0, 0]
    bm = jnp.max(l_row)
    m_old = stat_sc[0]
    m_new = jnp.maximum(m_old, bm)
    scale = jnp.exp(m_old - m_new)
    w_row = jnp.exp(l_row - m_new)
    stat_sc[0] = m_new
    stat_sc[1] = stat_sc[1] * scale + jnp.sum(w_row)
    w_col = w_row.reshape(BLK, 1)
    part = jnp.sum(blk * w_col, axis=0, keepdims=True)
    acc_sc[...] = acc_sc[...] * scale + part

    @pl.when(i == NB - 1)
    def _epilogue():
        state_row = acc_sc[...] / stat_sc[1]

        t = _bdot(state_row, w1_ref[...]) + b1_ref[...]
        value = _bdot(jnp.maximum(t, 0.0), w2_ref[...]) + b2_ref[0, 0]

        lane = jax.lax.broadcasted_iota(jnp.int32, (1, BLK), 1)

        def body(j, carry):
            m2, s2, gmax, gidx = carry
            blk2 = mem_sc[pl.ds(j * BLK, BLK), :]
            lg = _bdot(state_row, blk2)
            bmax = jnp.max(lg)
            new_m = jnp.maximum(m2, bmax)
            s2 = s2 * jnp.exp(m2 - new_m) + jnp.sum(jnp.exp(lg - new_m))
            barg = jnp.min(jnp.where(lg == bmax, lane, N))
            gidx = jnp.where(bmax > gmax, j * BLK + barg, gidx)
            gmax = jnp.maximum(gmax, bmax)
            return new_m, s2, gmax, gidx

        m2, s2, gmax, gidx = jax.lax.fori_loop(
            0, NB, body,
            (-jnp.inf, jnp.float32(0.0), -jnp.inf, jnp.int32(0)))
        lse = m2 + jnp.log(s2)
        nll_ref[...] = jnp.full((1, 1), lse - gmax, dtype=jnp.float32)
        val_ref[...] = value

        g = (gidx // 8) * 8
        tile = mem_sc[pl.ds(g, 8), :]
        rows8 = jax.lax.broadcasted_iota(jnp.int32, (8, 1), 0)
        act_row = jnp.sum(jnp.where(rows8 == (gidx - g), tile,
                                    jnp.bfloat16(0.0)),
                          axis=0, keepdims=True)

        gi = _bdot(act_row, wih_ref[...]) + bih_ref[...]
        gh = _bdot(state_row, whh_ref[...]) + bhh_ref[...]
        i_r, i_z, i_n = gi[:, 0:D], gi[:, D:2 * D], gi[:, 2 * D:3 * D]
        h_r, h_z, h_n = gh[:, 0:D], gh[:, D:2 * D], gh[:, 2 * D:3 * D]
        r = _sigmoid(i_r + h_r)
        z = _sigmoid(i_z + h_z)
        n = jnp.tanh(i_n + r * h_n)
        state_ref[...] = (1.0 - z) * n + z * state_row


def kernel(mem, Wa, ba, W1, b1, W2, b2, W_ih, b_ih, W_hh, b_hh):
    ba2 = ba.reshape(1, 1)
    b1r = b1.reshape(1, D)
    b2_2 = b2.reshape(1, 1)
    bihr = b_ih.reshape(1, 3 * D)
    bhhr = b_hh.reshape(1, 3 * D)

    const = lambda i: (0, 0)
    nll, val, st = pl.pallas_call(
        _body,
        grid=(NB,),
        in_specs=[
            pl.BlockSpec((BLK, D), lambda i: (i, 0)),
            pl.BlockSpec((1, D), const),
            pl.BlockSpec((1, 1), const),
            pl.BlockSpec((D, D), const),
            pl.BlockSpec((1, D), const),
            pl.BlockSpec((1, D), const),
            pl.BlockSpec((1, 1), const),
            pl.BlockSpec((3 * D, D), const),
            pl.BlockSpec((1, 3 * D), const),
            pl.BlockSpec((3 * D, D), const),
            pl.BlockSpec((1, 3 * D), const),
        ],
        out_specs=[
            pl.BlockSpec((1, 1), const),
            pl.BlockSpec((1, 1), const),
            pl.BlockSpec((1, D), const),
        ],
        out_shape=[
            jax.ShapeDtypeStruct((1, 1), jnp.float32),
            jax.ShapeDtypeStruct((1, 1), jnp.float32),
            jax.ShapeDtypeStruct((1, D), jnp.float32),
        ],
        scratch_shapes=[
            pltpu.VMEM((N, D), jnp.bfloat16),
            pltpu.VMEM((1, D), jnp.float32),
            pltpu.SMEM((2,), jnp.float32),
        ],
        compiler_params=pltpu.CompilerParams(
            dimension_semantics=("arbitrary",),
            vmem_limit_bytes=64 * 1024 * 1024,
        ),
    )(mem, Wa, ba2, W1, b1r, W2, b2_2, W_ih, bihr, W_hh, bhhr)
    return nll.reshape(()), val, st


# speedup vs baseline: 2.6160x; 1.0060x over previous
import jax
import jax.numpy as jnp
from jax import lax
from jax.experimental import pallas as pl
from jax.experimental.pallas import tpu as pltpu

N = 100000
D = 128
BLK = 10000
NB = N // BLK

_DOT_T = (((1,), (1,)), ((), ()))


def _sigmoid(x):
    return 1.0 / (1.0 + jnp.exp(-x))


def _bdot(a, b):
    return lax.dot_general(a.astype(jnp.bfloat16), b.astype(jnp.bfloat16),
                           _DOT_T, preferred_element_type=jnp.float32)


def _body(mem_ref, wa_ref, ba_ref, w1_ref, b1_ref, w2_ref, b2_ref,
          wih_ref, bih_ref, whh_ref, bhh_ref,
          nll_ref, val_ref, state_ref,
          mem_sc, acc_sc, stat_sc):
    i = pl.program_id(0)

    @pl.when(i == 0)
    def _init():
        stat_sc[0] = -jnp.inf
        stat_sc[1] = 0.0
        acc_sc[...] = jnp.zeros_like(acc_sc)

    blk = mem_ref[...]
    blk16 = blk.astype(jnp.bfloat16)
    mem_sc[pl.ds(i * BLK, BLK), :] = blk16

    l_row = _bdot(wa_ref[...], blk16) + ba_ref[0, 0]
    bm = jnp.max(l_row)
    m_old = stat_sc[0]
    m_new = jnp.maximum(m_old, bm)
    scale = jnp.exp(m_old - m_new)
    w_row = jnp.exp(l_row - m_new)
    stat_sc[0] = m_new
    stat_sc[1] = stat_sc[1] * scale + jnp.sum(w_row)
    w_col = w_row.reshape(BLK, 1)
    part = jnp.sum(blk * w_col, axis=0, keepdims=True)
    acc_sc[...] = acc_sc[...] * scale + part

    @pl.when(i == NB - 1)
    def _epilogue():
        state_row = acc_sc[...] / stat_sc[1]

        t = _bdot(state_row, w1_ref[...]) + b1_ref[...]
        value = _bdot(jnp.maximum(t, 0.0), w2_ref[...]) + b2_ref[0, 0]

        lane = jax.lax.broadcasted_iota(jnp.int32, (1, BLK), 1)

        def body(j, carry):
            m2, s2, gmax, gidx = carry
            blk2 = mem_sc[pl.ds(j * BLK, BLK), :]
            lg = _bdot(state_row, blk2)
            bmax = jnp.max(lg)
            new_m = jnp.maximum(m2, bmax)
            s2 = s2 * jnp.exp(m2 - new_m) + jnp.sum(jnp.exp(lg - new_m))
            barg = jnp.min(jnp.where(lg == bmax, lane, N))
            gidx = jnp.where(bmax > gmax, j * BLK + barg, gidx)
            gmax = jnp.maximum(gmax, bmax)
            return new_m, s2, gmax, gidx

        m2, s2, gmax, gidx = jax.lax.fori_loop(
            0, NB, body,
            (-jnp.inf, jnp.float32(0.0), -jnp.inf, jnp.int32(0)))
        lse = m2 + jnp.log(s2)
        nll_ref[...] = jnp.full((1, 1), lse - gmax, dtype=jnp.float32)
        val_ref[...] = value

        g = pl.multiple_of((gidx // 16) * 16, 16)
        tile = mem_sc[pl.ds(g, 16), :]
        rows16 = jax.lax.broadcasted_iota(jnp.int32, (16, 1), 0)
        act_row = jnp.sum(jnp.where(rows16 == (gidx - g), tile,
                                    jnp.bfloat16(0.0)),
                          axis=0, keepdims=True)

        gi = _bdot(act_row, wih_ref[...]) + bih_ref[...]
        gh = _bdot(state_row, whh_ref[...]) + bhh_ref[...]
        i_r, i_z, i_n = gi[:, 0:D], gi[:, D:2 * D], gi[:, 2 * D:3 * D]
        h_r, h_z, h_n = gh[:, 0:D], gh[:, D:2 * D], gh[:, 2 * D:3 * D]
        r = _sigmoid(i_r + h_r)
        z = _sigmoid(i_z + h_z)
        n = jnp.tanh(i_n + r * h_n)
        state_ref[...] = (1.0 - z) * n + z * state_row


def kernel(mem, Wa, ba, W1, b1, W2, b2, W_ih, b_ih, W_hh, b_hh):
    ba2 = ba.reshape(1, 1)
    b1r = b1.reshape(1, D)
    b2_2 = b2.reshape(1, 1)
    bihr = b_ih.reshape(1, 3 * D)
    bhhr = b_hh.reshape(1, 3 * D)

    const = lambda i: (0, 0)
    nll, val, st = pl.pallas_call(
        _body,
        grid=(NB,),
        in_specs=[
            pl.BlockSpec((BLK, D), lambda i: (i, 0)),
            pl.BlockSpec((1, D), const),
            pl.BlockSpec((1, 1), const),
            pl.BlockSpec((D, D), const),
            pl.BlockSpec((1, D), const),
            pl.BlockSpec((1, D), const),
            pl.BlockSpec((1, 1), const),
            pl.BlockSpec((3 * D, D), const),
            pl.BlockSpec((1, 3 * D), const),
            pl.BlockSpec((3 * D, D), const),
            pl.BlockSpec((1, 3 * D), const),
        ],
        out_specs=[
            pl.BlockSpec((1, 1), const),
            pl.BlockSpec((1, 1), const),
            pl.BlockSpec((1, D), const),
        ],
        out_shape=[
            jax.ShapeDtypeStruct((1, 1), jnp.float32),
            jax.ShapeDtypeStruct((1, 1), jnp.float32),
            jax.ShapeDtypeStruct((1, D), jnp.float32),
        ],
        scratch_shapes=[
            pltpu.VMEM((N, D), jnp.bfloat16),
            pltpu.VMEM((1, D), jnp.float32),
            pltpu.SMEM((2,), jnp.float32),
        ],
        compiler_params=pltpu.CompilerParams(
            dimension_semantics=("arbitrary",),
            vmem_limit_bytes=64 * 1024 * 1024,
        ),
    )(mem, Wa, ba2, W1, b1r, W2, b2_2, W_ih, bihr, W_hh, bhhr)
    return nll.reshape(()), val, st
